# baseline (speedup 1.0000x reference)
import functools

import jax
import jax.numpy as jnp
from jax.experimental import pallas as pl
from jax.experimental.pallas import tpu as pltpu

_MARGIN = 0.2
_EPS = 1e-12


def _normalize(x, eps):
    n = jnp.sqrt(jnp.sum(x * x, axis=1, keepdims=True))
    return x / jnp.maximum(n, eps)


def _triplet_body(a_ref, p_ref, pd_ref, out_ref, min_ref,
                  *, bm, cj, n, nj, margin, eps):
    i = pl.program_id(0)
    j = pl.program_id(1)

    a = _normalize(a_ref[...], eps)
    p = _normalize(p_ref[...], eps)

    ones_row = jnp.ones((1, a.shape[1]), jnp.float32)
    p_sq = jax.lax.dot_general(
        ones_row, p * p, (((1,), (1,)), ((), ())),
        preferred_element_type=jnp.float32)

    dots = jax.lax.dot_general(
        a.astype(jnp.bfloat16), p.astype(jnp.bfloat16),
        (((1,), (1,)), ((), ())),
        preferred_element_type=jnp.float32)

    vals = p_sq - 2.0 * dots

    @pl.when(j == 0)
    def _init_min():
        min_ref[...] = jnp.full((bm, 1), jnp.inf, jnp.float32)

    jd = (i * bm) // cj

    @pl.when(j == jd)
    def _masked_min():
        row_g = i * bm + jax.lax.broadcasted_iota(jnp.int32, (bm, cj), 0)
        col_g = j * cj + jax.lax.broadcasted_iota(jnp.int32, (bm, cj), 1)
        v = jnp.where(row_g == col_g, jnp.inf, vals)
        min_ref[...] = jnp.minimum(min_ref[...],
                                   jnp.min(v, axis=1, keepdims=True))

    @pl.when(j != jd)
    def _plain_min():
        min_ref[...] = jnp.minimum(min_ref[...],
                                   jnp.min(vals, axis=1, keepdims=True))

    @pl.when(j == nj - 1)
    def _finalize():
        pd = _normalize(pd_ref[...], eps)
        a_sq = jnp.sum(a * a, axis=1, keepdims=True)
        ap = jnp.sum((a - pd) * (a - pd), axis=1, keepdims=True)
        an = a_sq + min_ref[...]
        losses = jnp.maximum(ap - an + margin, 0.0)
        part = jnp.sum(losses, keepdims=True) * (1.0 / n)

        @pl.when(i == 0)
        def _init_out():
            out_ref[...] = jnp.zeros_like(out_ref)

        out_ref[...] += part


@jax.jit
def kernel(anchors, positives):
    n, d = anchors.shape
    bm = 256
    cj = 2048
    ni, nj = n // bm, n // cj
    body = functools.partial(_triplet_body, bm=bm, cj=cj, n=n, nj=nj,
                             margin=_MARGIN, eps=_EPS)
    out = pl.pallas_call(
        body,
        grid=(ni, nj),
        in_specs=[
            pl.BlockSpec((bm, d), lambda i, j: (i, 0)),
            pl.BlockSpec((cj, d), lambda i, j: (j, 0)),
            pl.BlockSpec((bm, d), lambda i, j: (i, 0)),
        ],
        out_specs=pl.BlockSpec((1, 1), lambda i, j: (0, 0)),
        out_shape=jax.ShapeDtypeStruct((1, 1), jnp.float32),
        scratch_shapes=[pltpu.VMEM((bm, 1), jnp.float32)],
    )(anchors, positives, positives)
    return out[0, 0]


# speedup vs baseline: 1.0512x; 1.0512x over previous
import functools

import jax
import jax.numpy as jnp
from jax.experimental import pallas as pl
from jax.experimental.pallas import tpu as pltpu

_MARGIN = 0.2
_EPS = 1e-12


def _normalize(x, eps):
    n = jnp.sqrt(jnp.sum(x * x, axis=1, keepdims=True))
    return x / jnp.maximum(n, eps)


def _fold_min(v, bm, cj):
    return jnp.min(v.reshape(bm, cj // 128, 128), axis=1)


def _triplet_body(a_ref, p_ref, pd_ref, out_ref,
                  paug_ref, aaug_ref, asq_ref, min_ref,
                  *, bm, cj, n, nj, d, margin, eps):
    i = pl.program_id(0)
    j = pl.program_id(1)

    @pl.when((i == 0) & (j == 0))
    def _build_paug():
        pn = _normalize(p_ref[...], eps)
        paug_ref[:, :d] = -2.0 * pn
        paug_ref[:, d:] = jnp.sum(pn * pn, axis=1, keepdims=True)

    @pl.when(j == 0)
    def _build_aaug():
        an_ = _normalize(a_ref[...], eps)
        aaug_ref[:, :d] = an_
        aaug_ref[:, d:] = jnp.ones((bm, 1), jnp.float32)
        asq_ref[...] = jnp.sum(an_ * an_, axis=1, keepdims=True)
        min_ref[...] = jnp.full((bm, 128), jnp.inf, jnp.float32)

    vals = jax.lax.dot_general(
        aaug_ref[...], paug_ref[pl.ds(j * cj, cj), :],
        (((1,), (1,)), ((), ())),
        preferred_element_type=jnp.float32)

    jd = (i * bm) // cj

    @pl.when(j == jd)
    def _masked_min():
        row_g = i * bm + jax.lax.broadcasted_iota(jnp.int32, (bm, cj), 0)
        col_g = j * cj + jax.lax.broadcasted_iota(jnp.int32, (bm, cj), 1)
        v = jnp.where(row_g == col_g, jnp.inf, vals)
        min_ref[...] = jnp.minimum(min_ref[...], _fold_min(v, bm, cj))

    @pl.when(j != jd)
    def _plain_min():
        min_ref[...] = jnp.minimum(min_ref[...], _fold_min(vals, bm, cj))

    @pl.when(j == nj - 1)
    def _finalize():
        rowmin = jnp.min(min_ref[...], axis=1, keepdims=True)
        an_dist = asq_ref[...] + rowmin
        a_n = aaug_ref[:, :d]
        pd = _normalize(pd_ref[...], eps)
        ap = jnp.sum((a_n - pd) * (a_n - pd), axis=1, keepdims=True)
        losses = jnp.maximum(ap - an_dist + margin, 0.0)
        part = jnp.sum(losses, keepdims=True) * (1.0 / n)

        @pl.when(i == 0)
        def _init_out():
            out_ref[...] = jnp.zeros_like(out_ref)

        out_ref[...] += part


@jax.jit
def kernel(anchors, positives):
    n, d = anchors.shape
    bm = 256
    cj = 2048
    ni, nj = n // bm, n // cj
    body = functools.partial(_triplet_body, bm=bm, cj=cj, n=n, nj=nj, d=d,
                             margin=_MARGIN, eps=_EPS)
    out = pl.pallas_call(
        body,
        grid=(ni, nj),
        in_specs=[
            pl.BlockSpec((bm, d), lambda i, j: (i, 0)),
            pl.BlockSpec((n, d), lambda i, j: (0, 0)),
            pl.BlockSpec((bm, d), lambda i, j: (i, 0)),
        ],
        out_specs=pl.BlockSpec((1, 1), lambda i, j: (0, 0)),
        out_shape=jax.ShapeDtypeStruct((1, 1), jnp.float32),
        scratch_shapes=[
            pltpu.VMEM((n, d + 1), jnp.float32),
            pltpu.VMEM((bm, d + 1), jnp.float32),
            pltpu.VMEM((bm, 1), jnp.float32),
            pltpu.VMEM((bm, 128), jnp.float32),
        ],
    )(anchors, positives, positives)
    return out[0, 0]


# speedup vs baseline: 1.7956x; 1.7082x over previous
import functools

import jax
import jax.numpy as jnp
from jax.experimental import pallas as pl
from jax.experimental.pallas import tpu as pltpu

_MARGIN = 0.2
_EPS = 1e-12


def _normalize(x, eps):
    n = jnp.sqrt(jnp.sum(x * x, axis=1, keepdims=True))
    return x / jnp.maximum(n, eps)


def _fold_min(v, bm, cj):
    parts = [v[:, k * 128:(k + 1) * 128] for k in range(cj // 128)]
    while len(parts) > 1:
        nxt = [jnp.minimum(parts[t], parts[t + 1])
               for t in range(0, len(parts) - 1, 2)]
        if len(parts) % 2:
            nxt.append(parts[-1])
        parts = nxt
    return parts[0]


def _triplet_body(a_ref, p_ref, pd_ref, out_ref,
                  paug_ref, aaug_ref, asq_ref, min_ref,
                  *, bm, cj, n, nj, d, margin, eps):
    i = pl.program_id(0)
    j = pl.program_id(1)

    @pl.when((i == 0) & (j == 0))
    def _build_paug():
        pn = _normalize(p_ref[...], eps)
        paug_ref[:, :d] = -2.0 * pn
        paug_ref[:, d:] = jnp.sum(pn * pn, axis=1, keepdims=True)

    @pl.when(j == 0)
    def _build_aaug():
        an_ = _normalize(a_ref[...], eps)
        aaug_ref[:, :d] = an_
        aaug_ref[:, d:] = jnp.ones((bm, 1), jnp.float32)
        asq_ref[...] = jnp.sum(an_ * an_, axis=1, keepdims=True)
        min_ref[...] = jnp.full((bm, 128), jnp.inf, jnp.float32)

    vals = jax.lax.dot_general(
        aaug_ref[...], paug_ref[pl.ds(j * cj, cj), :],
        (((1,), (1,)), ((), ())),
        preferred_element_type=jnp.float32)

    jd = (i * bm) // cj

    @pl.when(j == jd)
    def _masked_min():
        row_g = i * bm + jax.lax.broadcasted_iota(jnp.int32, (bm, cj), 0)
        col_g = j * cj + jax.lax.broadcasted_iota(jnp.int32, (bm, cj), 1)
        v = jnp.where(row_g == col_g, jnp.inf, vals)
        min_ref[...] = jnp.minimum(min_ref[...], _fold_min(v, bm, cj))

    @pl.when(j != jd)
    def _plain_min():
        min_ref[...] = jnp.minimum(min_ref[...], _fold_min(vals, bm, cj))

    @pl.when(j == nj - 1)
    def _finalize():
        rowmin = jnp.min(min_ref[...], axis=1, keepdims=True)
        an_dist = asq_ref[...] + rowmin
        a_n = aaug_ref[:, :d]
        pd = _normalize(pd_ref[...], eps)
        ap = jnp.sum((a_n - pd) * (a_n - pd), axis=1, keepdims=True)
        losses = jnp.maximum(ap - an_dist + margin, 0.0)
        part = jnp.sum(losses, keepdims=True) * (1.0 / n)

        @pl.when(i == 0)
        def _init_out():
            out_ref[...] = jnp.zeros_like(out_ref)

        out_ref[...] += part


@jax.jit
def kernel(anchors, positives):
    n, d = anchors.shape
    bm = 256
    cj = 2048
    ni, nj = n // bm, n // cj
    body = functools.partial(_triplet_body, bm=bm, cj=cj, n=n, nj=nj, d=d,
                             margin=_MARGIN, eps=_EPS)
    out = pl.pallas_call(
        body,
        grid=(ni, nj),
        in_specs=[
            pl.BlockSpec((bm, d), lambda i, j: (i, 0)),
            pl.BlockSpec((n, d), lambda i, j: (0, 0)),
            pl.BlockSpec((bm, d), lambda i, j: (i, 0)),
        ],
        out_specs=pl.BlockSpec((1, 1), lambda i, j: (0, 0)),
        out_shape=jax.ShapeDtypeStruct((1, 1), jnp.float32),
        scratch_shapes=[
            pltpu.VMEM((n, d + 1), jnp.float32),
            pltpu.VMEM((bm, d + 1), jnp.float32),
            pltpu.VMEM((bm, 1), jnp.float32),
            pltpu.VMEM((bm, 128), jnp.float32),
        ],
    )(anchors, positives, positives)
    return out[0, 0]
